# Initial kernel scaffold; baseline (speedup 1.0000x reference)
import jax
import jax.numpy as jnp
from jax.experimental import pallas as pl


def kernel(pred, gt):
    raise NotImplementedError("write your pallas kernel here")


# speedup vs baseline: 12.2176x; 12.2176x over previous
import functools

import jax
import jax.numpy as jnp
from jax import lax
from jax.experimental import pallas as pl
from jax.experimental.pallas import tpu as pltpu, tpu_sc as plsc

N = 16 * 1 * 512 * 512
K = int(N * 0.3)
NBIN = 1024
LOSS_MAX = 100.0
C_SCALE = float(NBIN) / LOSS_MAX
W_COARSE = LOSS_MAX / NBIN

NW = 32
PER_W = N // NW
CHUNK = 8192
NCHUNK = PER_W // CHUNK
GROUPS = CHUNK // 16

_MESH = plsc.VectorSubcoreMesh(core_axis_name="c", subcore_axis_name="s")


def _loss_body(pred_ref, gt_ref, loss_ref):
    p = pred_ref[...]
    g = gt_ref[...]
    log_p = jnp.maximum(jnp.log(p), -100.0)
    log_1mp = jnp.maximum(jnp.log(1.0 - p), -100.0)
    loss_ref[...] = -(g * log_p + (1.0 - g) * log_1mp)


def _bce_loss(pred2d, gt2d):
    rows = pred2d.shape[0]
    blk = rows // 8
    return pl.pallas_call(
        _loss_body,
        grid=(8,),
        in_specs=[pl.BlockSpec((blk, 1024), lambda i: (i, 0)),
                  pl.BlockSpec((blk, 1024), lambda i: (i, 0))],
        out_specs=pl.BlockSpec((blk, 1024), lambda i: (i, 0)),
        out_shape=jax.ShapeDtypeStruct((rows, 1024), jnp.float32),
    )(pred2d, gt2d)


def _hist_body(loss_hbm, lo_hbm, invw_hbm, bsel_hbm,
               cnt_hbm, sum_hbm,
               buf, prm_f, prm_i, hc, hs, oc, os_):
    wid = lax.axis_index("s") * 2 + lax.axis_index("c")
    base = wid * PER_W

    pltpu.sync_copy(lo_hbm, prm_f.at[0])
    pltpu.sync_copy(invw_hbm, prm_f.at[1])
    pltpu.sync_copy(bsel_hbm, prm_i.at[0])
    lo = prm_f[0]
    invw = prm_f[1]
    bsel = prm_i[0]

    zi = jnp.zeros((16,), jnp.int32)
    zf = jnp.zeros((16,), jnp.float32)

    def zero(g, _):
        hc[pl.ds(g * 16, 16)] = zi
        hs[pl.ds(g * 16, 16)] = zf
        return _
    lax.fori_loop(0, NBIN, zero, None)

    lane_off = lax.iota(jnp.int32, 16) * NBIN
    ones_i = jnp.ones((16,), jnp.int32)

    def chunk_step(c, _):
        pltpu.sync_copy(loss_hbm.at[pl.ds(base + c * CHUNK, CHUNK)], buf)

        def group(g, _):
            v = buf[pl.ds(g * 16, 16)]
            cidx = jnp.clip((v * C_SCALE).astype(jnp.int32), 0, NBIN - 1)
            mask = (bsel < 0) | (cidx == bsel)
            fidx = jnp.clip(((v - lo) * invw).astype(jnp.int32), 0, NBIN - 1)
            addr = fidx + lane_off
            plsc.addupdate_scatter(hc, [addr], ones_i, mask=mask)
            plsc.addupdate_scatter(hs, [addr], v, mask=mask)
            return _
        lax.fori_loop(0, GROUPS, group, None)
        return _
    lax.fori_loop(0, NCHUNK, chunk_step, None)

    def red(g, _):
        ac = hc[pl.ds(g * 16, 16)]
        af = hs[pl.ds(g * 16, 16)]
        for l in range(1, 16):
            ac = ac + hc[pl.ds(l * NBIN + g * 16, 16)]
            af = af + hs[pl.ds(l * NBIN + g * 16, 16)]
        oc[pl.ds(g * 16, 16)] = ac
        os_[pl.ds(g * 16, 16)] = af
        return _
    lax.fori_loop(0, NBIN // 16, red, None)

    pltpu.sync_copy(oc, cnt_hbm.at[wid])
    pltpu.sync_copy(os_, sum_hbm.at[wid])


@functools.partial(
    pl.kernel,
    mesh=_MESH,
    compiler_params=pltpu.CompilerParams(needs_layout_passes=False),
    out_type=[jax.ShapeDtypeStruct((NW, NBIN), jnp.int32),
              jax.ShapeDtypeStruct((NW, NBIN), jnp.float32)],
    scratch_types=[
        pltpu.VMEM((CHUNK,), jnp.float32),
        pltpu.VMEM((2, 16), jnp.float32),
        pltpu.VMEM((1, 16), jnp.int32),
        pltpu.VMEM((16 * NBIN,), jnp.int32),
        pltpu.VMEM((16 * NBIN,), jnp.float32),
        pltpu.VMEM((NBIN,), jnp.int32),
        pltpu.VMEM((NBIN,), jnp.float32),
    ],
)
def _sc_hist(loss_hbm, lo_hbm, invw_hbm, bsel_hbm, cnt_hbm, sum_hbm,
             buf, prm_f, prm_i, hc, hs, oc, os_):
    _hist_body(loss_hbm, lo_hbm, invw_hbm, bsel_hbm, cnt_hbm, sum_hbm,
               buf, prm_f, prm_i, hc, hs, oc, os_)


def _suffix_sum(x):
    row = lax.broadcasted_iota(jnp.int32, (NBIN, NBIN), 0)
    col = lax.broadcasted_iota(jnp.int32, (NBIN, NBIN), 1)
    tri = (row >= col).astype(jnp.float32)
    return jnp.dot(x.reshape(1, NBIN), tri,
                   preferred_element_type=jnp.float32).reshape(NBIN)


def _select_body(cnt_ref, sum_ref, out_ref):
    c = jnp.sum(cnt_ref[...], axis=0)
    s = jnp.sum(sum_ref[...], axis=0)
    cg = _suffix_sum(c.astype(jnp.float32))
    bsel = jnp.sum((cg >= K).astype(jnp.int32)) - 1
    bins = lax.iota(jnp.int32, NBIN)
    above = bins > bsel
    c_above = jnp.sum(jnp.where(above, c, 0))
    s_above = jnp.sum(jnp.where(above, s, 0.0))
    total = jnp.sum(s)
    vals = [bsel.astype(jnp.float32) * W_COARSE,
            jnp.float32(NBIN / W_COARSE),
            bsel.astype(jnp.float32),
            c_above.astype(jnp.float32),
            s_above,
            total]
    row = lax.broadcasted_iota(jnp.int32, (8, 128), 0)
    col = lax.broadcasted_iota(jnp.int32, (8, 128), 1)
    o = jnp.zeros((8, 128), jnp.float32)
    for j, v in enumerate(vals):
        o = jnp.where((row == 0) & (col == j), v, o)
    out_ref[...] = o


def _select(cnt, sums):
    return pl.pallas_call(
        _select_body,
        out_shape=jax.ShapeDtypeStruct((8, 128), jnp.float32),
    )(cnt, sums)


def _final_body(prm_ref, cnt_ref, sum_ref, out_ref):
    lo = prm_ref[0, 0]
    c_above = prm_ref[0, 3]
    s_above = prm_ref[0, 4]
    total = prm_ref[0, 5]
    fc = jnp.sum(cnt_ref[...], axis=0)
    fs = jnp.sum(sum_ref[...], axis=0)
    cgf = _suffix_sum(fc.astype(jnp.float32))
    fsel = jnp.sum((c_above + cgf >= K).astype(jnp.int32)) - 1
    bins = lax.iota(jnp.int32, NBIN)
    above = bins > fsel
    n_above_f = jnp.sum(jnp.where(above, fc, 0)).astype(jnp.float32)
    s_above_f = jnp.sum(jnp.where(above, fs, 0.0))
    needed = K - c_above - n_above_f
    w_f = W_COARSE / NBIN
    t_est = lo + (fsel.astype(jnp.float32) + 0.5) * w_f
    topk_sum = s_above + s_above_f + needed * t_est
    loss_total = total / (N + 1e-12) + topk_sum / K
    out_ref[...] = jnp.full((1, 1), loss_total)


def _final(prm, cnt, sums):
    return pl.pallas_call(
        _final_body,
        out_shape=jax.ShapeDtypeStruct((1, 1), jnp.float32),
    )(prm, cnt, sums)


def kernel(pred, gt):
    pred2d = pred.reshape(4096, 1024)
    gt2d = gt.reshape(4096, 1024)
    loss = _bce_loss(pred2d, gt2d)
    loss_flat = loss.reshape(N)

    lo0 = jnp.zeros((16,), jnp.float32)
    invw0 = jnp.full((16,), C_SCALE, jnp.float32)
    bsel0 = jnp.full((16,), -1, jnp.int32)
    cc, cs = _sc_hist(loss_flat, lo0, invw0, bsel0)

    prm = _select(cc, cs)

    lo1 = jnp.full((16,), prm[0, 0])
    invw1 = jnp.full((16,), prm[0, 1])
    bsel1 = jnp.full((16,), prm[0, 2].astype(jnp.int32))
    fc, fs = _sc_hist(loss_flat, lo1, invw1, bsel1)

    out = _final(prm, fc, fs)
    return out[0, 0]


# speedup vs baseline: 26.9856x; 2.2087x over previous
import functools

import jax
import jax.numpy as jnp
from jax import lax
from jax.experimental import pallas as pl
from jax.experimental.pallas import tpu as pltpu, tpu_sc as plsc

N = 16 * 1 * 512 * 512
K = int(N * 0.3)
NBIN = 1024
LOSS_MAX = 100.0
C_SCALE = float(NBIN) / LOSS_MAX
W_COARSE = LOSS_MAX / NBIN

NW = 32
PER_W = N // NW
CHUNK = 8192
NCHUNK = PER_W // CHUNK
GROUPS = CHUNK // 16

_MESH = plsc.VectorSubcoreMesh(core_axis_name="c", subcore_axis_name="s")


def _loss_body(pred_ref, gt_ref, loss_ref):
    p = pred_ref[...]
    g = gt_ref[...]
    log_p = jnp.maximum(jnp.log(p), -100.0)
    log_1mp = jnp.maximum(jnp.log(1.0 - p), -100.0)
    loss_ref[...] = -(g * log_p + (1.0 - g) * log_1mp)


def _bce_loss(pred2d, gt2d):
    rows = pred2d.shape[0]
    blk = rows // 8
    return pl.pallas_call(
        _loss_body,
        grid=(8,),
        in_specs=[pl.BlockSpec((blk, 1024), lambda i: (i, 0)),
                  pl.BlockSpec((blk, 1024), lambda i: (i, 0))],
        out_specs=pl.BlockSpec((blk, 1024), lambda i: (i, 0)),
        out_shape=jax.ShapeDtypeStruct((rows, 1024), jnp.float32),
    )(pred2d, gt2d)


def _hist_common(masked, loss_hbm, prm, cnt_hbm, sum_hbm,
                 buf0, buf1, prm_f, prm_i, hc, hs, oc, os_, sem0, sem1):
    wid = lax.axis_index("s") * 2 + lax.axis_index("c")
    base = wid * PER_W

    if masked:
        lo_hbm, invw_hbm, bsel_hbm = prm
        pltpu.sync_copy(lo_hbm, prm_f.at[0])
        pltpu.sync_copy(invw_hbm, prm_f.at[1])
        pltpu.sync_copy(bsel_hbm, prm_i.at[0])
        lo = prm_f[0]
        invw = prm_f[1]
        bsel = prm_i[0]

    zi = jnp.zeros((16,), jnp.int32)
    zf = jnp.zeros((16,), jnp.float32)

    @plsc.parallel_loop(0, NBIN, unroll=8)
    def _zero(g):
        hc[pl.ds(g * 16, 16)] = zi
        hs[pl.ds(g * 16, 16)] = zf

    lane_off = lax.iota(jnp.int32, 16) * NBIN
    ones_i = jnp.ones((16,), jnp.int32)

    bufs = (buf0, buf1)
    sems = (sem0, sem1)
    pend = [None, None]
    pend[0] = pltpu.async_copy(loss_hbm.at[pl.ds(base, CHUNK)], buf0, sem0)
    for c in range(NCHUNK):
        pend[c % 2].wait()
        if c + 1 < NCHUNK:
            pend[(c + 1) % 2] = pltpu.async_copy(
                loss_hbm.at[pl.ds(base + (c + 1) * CHUNK, CHUNK)],
                bufs[(c + 1) % 2], sems[(c + 1) % 2])
        buf = bufs[c % 2]

        @plsc.parallel_loop(0, GROUPS, unroll=8)
        def _group(g):
            v = buf[pl.ds(g * 16, 16)]
            if masked:
                cidx = jnp.clip((v * C_SCALE).astype(jnp.int32), 0, NBIN - 1)
                mask = cidx == bsel
                fidx = jnp.clip(((v - lo) * invw).astype(jnp.int32),
                                0, NBIN - 1)
                addr = fidx + lane_off
                plsc.addupdate_scatter(hc, [addr], ones_i, mask=mask)
                plsc.addupdate_scatter(hs, [addr], v, mask=mask)
            else:
                addr = jnp.clip((v * C_SCALE).astype(jnp.int32),
                                0, NBIN - 1) + lane_off
                plsc.addupdate_scatter(hc, [addr], ones_i)
                plsc.addupdate_scatter(hs, [addr], v)

    @plsc.parallel_loop(0, NBIN // 16, unroll=2)
    def _red(g):
        ac = hc[pl.ds(g * 16, 16)]
        af = hs[pl.ds(g * 16, 16)]
        for l in range(1, 16):
            ac = ac + hc[pl.ds(l * NBIN + g * 16, 16)]
            af = af + hs[pl.ds(l * NBIN + g * 16, 16)]
        oc[pl.ds(g * 16, 16)] = ac
        os_[pl.ds(g * 16, 16)] = af

    pltpu.sync_copy(oc, cnt_hbm.at[wid])
    pltpu.sync_copy(os_, sum_hbm.at[wid])


_SC_OUT = [jax.ShapeDtypeStruct((NW, NBIN), jnp.int32),
           jax.ShapeDtypeStruct((NW, NBIN), jnp.float32)]
_SC_SCRATCH = [
    pltpu.VMEM((CHUNK,), jnp.float32),
    pltpu.VMEM((CHUNK,), jnp.float32),
    pltpu.VMEM((2, 16), jnp.float32),
    pltpu.VMEM((1, 16), jnp.int32),
    pltpu.VMEM((16 * NBIN,), jnp.int32),
    pltpu.VMEM((16 * NBIN,), jnp.float32),
    pltpu.VMEM((NBIN,), jnp.int32),
    pltpu.VMEM((NBIN,), jnp.float32),
    pltpu.SemaphoreType.DMA,
    pltpu.SemaphoreType.DMA,
]


@functools.partial(
    pl.kernel,
    mesh=_MESH,
    compiler_params=pltpu.CompilerParams(needs_layout_passes=False),
    out_type=_SC_OUT,
    scratch_types=_SC_SCRATCH,
)
def _sc_hist_coarse(loss_hbm, cnt_hbm, sum_hbm, *rest):
    _hist_common(False, loss_hbm, None, cnt_hbm, sum_hbm, *rest)


@functools.partial(
    pl.kernel,
    mesh=_MESH,
    compiler_params=pltpu.CompilerParams(needs_layout_passes=False),
    out_type=_SC_OUT,
    scratch_types=_SC_SCRATCH,
)
def _sc_hist_fine(loss_hbm, lo_hbm, invw_hbm, bsel_hbm, cnt_hbm, sum_hbm,
                  *rest):
    _hist_common(True, loss_hbm, (lo_hbm, invw_hbm, bsel_hbm),
                 cnt_hbm, sum_hbm, *rest)


def _suffix_sum(x):
    row = lax.broadcasted_iota(jnp.int32, (NBIN, NBIN), 0)
    col = lax.broadcasted_iota(jnp.int32, (NBIN, NBIN), 1)
    tri = (row >= col).astype(jnp.float32)
    return jnp.dot(x.reshape(1, NBIN), tri,
                   preferred_element_type=jnp.float32).reshape(NBIN)


def _select_body(cnt_ref, sum_ref, out_ref):
    c = jnp.sum(cnt_ref[...], axis=0)
    s = jnp.sum(sum_ref[...], axis=0)
    cg = _suffix_sum(c.astype(jnp.float32))
    bsel = jnp.sum((cg >= K).astype(jnp.int32)) - 1
    bins = lax.iota(jnp.int32, NBIN)
    above = bins > bsel
    c_above = jnp.sum(jnp.where(above, c, 0))
    s_above = jnp.sum(jnp.where(above, s, 0.0))
    total = jnp.sum(s)
    vals = [bsel.astype(jnp.float32) * W_COARSE,
            jnp.float32(NBIN / W_COARSE),
            bsel.astype(jnp.float32),
            c_above.astype(jnp.float32),
            s_above,
            total]
    row = lax.broadcasted_iota(jnp.int32, (8, 128), 0)
    col = lax.broadcasted_iota(jnp.int32, (8, 128), 1)
    o = jnp.zeros((8, 128), jnp.float32)
    for j, v in enumerate(vals):
        o = jnp.where((row == 0) & (col == j), v, o)
    out_ref[...] = o


def _select(cnt, sums):
    return pl.pallas_call(
        _select_body,
        out_shape=jax.ShapeDtypeStruct((8, 128), jnp.float32),
    )(cnt, sums)


def _final_body(prm_ref, cnt_ref, sum_ref, out_ref):
    lo = prm_ref[0, 0]
    c_above = prm_ref[0, 3]
    s_above = prm_ref[0, 4]
    total = prm_ref[0, 5]
    fc = jnp.sum(cnt_ref[...], axis=0)
    fs = jnp.sum(sum_ref[...], axis=0)
    cgf = _suffix_sum(fc.astype(jnp.float32))
    fsel = jnp.sum((c_above + cgf >= K).astype(jnp.int32)) - 1
    bins = lax.iota(jnp.int32, NBIN)
    above = bins > fsel
    n_above_f = jnp.sum(jnp.where(above, fc, 0)).astype(jnp.float32)
    s_above_f = jnp.sum(jnp.where(above, fs, 0.0))
    needed = K - c_above - n_above_f
    w_f = W_COARSE / NBIN
    t_est = lo + (fsel.astype(jnp.float32) + 0.5) * w_f
    topk_sum = s_above + s_above_f + needed * t_est
    loss_total = total / (N + 1e-12) + topk_sum / K
    out_ref[...] = jnp.full((1, 1), loss_total)


def _final(prm, cnt, sums):
    return pl.pallas_call(
        _final_body,
        out_shape=jax.ShapeDtypeStruct((1, 1), jnp.float32),
    )(prm, cnt, sums)


def kernel(pred, gt):
    pred2d = pred.reshape(4096, 1024)
    gt2d = gt.reshape(4096, 1024)
    loss = _bce_loss(pred2d, gt2d)
    loss_flat = loss.reshape(N)

    cc, cs = _sc_hist_coarse(loss_flat)

    prm = _select(cc, cs)

    lo1 = jnp.full((16,), prm[0, 0])
    invw1 = jnp.full((16,), prm[0, 1])
    bsel1 = jnp.full((16,), prm[0, 2].astype(jnp.int32))
    fc, fs = _sc_hist_fine(loss_flat, lo1, invw1, bsel1)

    out = _final(prm, fc, fs)
    return out[0, 0]


# speedup vs baseline: 34.4678x; 1.2773x over previous
import functools

import jax
import jax.numpy as jnp
from jax import lax
from jax.experimental import pallas as pl
from jax.experimental.pallas import tpu as pltpu, tpu_sc as plsc

N = 16 * 1 * 512 * 512
K = int(N * 0.3)
NBIN = 1024
LOSS_MAX = 100.0
C_SCALE = float(NBIN) / LOSS_MAX
W_COARSE = LOSS_MAX / NBIN

NW = 32
PER_W = N // NW
CHUNK = 8192
NCHUNK = PER_W // CHUNK
GROUPS = CHUNK // 16

_MESH = plsc.VectorSubcoreMesh(core_axis_name="c", subcore_axis_name="s")


def _loss_body(pred_ref, gt_ref, loss_ref):
    p = pred_ref[...]
    g = gt_ref[...]
    log_p = jnp.maximum(jnp.log(p), -100.0)
    log_1mp = jnp.maximum(jnp.log(1.0 - p), -100.0)
    loss_ref[...] = -(g * log_p + (1.0 - g) * log_1mp)


def _bce_loss(pred2d, gt2d):
    rows = pred2d.shape[0]
    blk = rows // 8
    return pl.pallas_call(
        _loss_body,
        grid=(8,),
        in_specs=[pl.BlockSpec((blk, 512), lambda i: (i, 0)),
                  pl.BlockSpec((blk, 512), lambda i: (i, 0))],
        out_specs=pl.BlockSpec((blk, 512), lambda i: (i, 0)),
        out_shape=jax.ShapeDtypeStruct((rows, 512), jnp.float32),
    )(pred2d, gt2d)


def _hist_common(masked, loss_hbm, prm, cnt_hbm, sum_hbm,
                 buf0, buf1, prm_f, prm_i, hc, hs, oc, os_, sem0, sem1):
    wid = lax.axis_index("s") * 2 + lax.axis_index("c")
    base = wid * PER_W

    if masked:
        lo_hbm, invw_hbm, bsel_hbm = prm
        pltpu.sync_copy(lo_hbm, prm_f.at[0])
        pltpu.sync_copy(invw_hbm, prm_f.at[1])
        pltpu.sync_copy(bsel_hbm, prm_i.at[0])
        lo = prm_f[0]
        invw = prm_f[1]
        bsel = prm_i[0]

    zi = jnp.zeros((16,), jnp.int32)
    zf = jnp.zeros((16,), jnp.float32)

    @plsc.parallel_loop(0, NBIN, unroll=8)
    def _zero(g):
        hc[pl.ds(g * 16, 16)] = zi
        hs[pl.ds(g * 16, 16)] = zf

    lane_off = lax.iota(jnp.int32, 16) * NBIN
    ones_i = jnp.ones((16,), jnp.int32)

    bufs = (buf0, buf1)
    sems = (sem0, sem1)
    pend = [None, None]
    pend[0] = pltpu.async_copy(loss_hbm.at[pl.ds(base, CHUNK)], buf0, sem0)
    for c in range(NCHUNK):
        pend[c % 2].wait()
        if c + 1 < NCHUNK:
            pend[(c + 1) % 2] = pltpu.async_copy(
                loss_hbm.at[pl.ds(base + (c + 1) * CHUNK, CHUNK)],
                bufs[(c + 1) % 2], sems[(c + 1) % 2])
        buf = bufs[c % 2]

        @plsc.parallel_loop(0, GROUPS, unroll=8)
        def _group(g):
            v = buf[pl.ds(g * 16, 16)]
            if masked:
                cidx = jnp.clip((v * C_SCALE).astype(jnp.int32), 0, NBIN - 1)
                mask = cidx == bsel
                fidx = jnp.clip(((v - lo) * invw).astype(jnp.int32),
                                0, NBIN - 1)
                addr = fidx + lane_off
                plsc.addupdate_scatter(hc, [addr], ones_i, mask=mask)
                plsc.addupdate_scatter(hs, [addr], v, mask=mask)
            else:
                addr = jnp.clip((v * C_SCALE).astype(jnp.int32),
                                0, NBIN - 1) + lane_off
                plsc.addupdate_scatter(hc, [addr], ones_i)
                plsc.addupdate_scatter(hs, [addr], v)

    @plsc.parallel_loop(0, NBIN // 16, unroll=2)
    def _red(g):
        ac = hc[pl.ds(g * 16, 16)]
        af = hs[pl.ds(g * 16, 16)]
        for l in range(1, 16):
            ac = ac + hc[pl.ds(l * NBIN + g * 16, 16)]
            af = af + hs[pl.ds(l * NBIN + g * 16, 16)]
        oc[pl.ds(g * 16, 16)] = ac
        os_[pl.ds(g * 16, 16)] = af

    pltpu.sync_copy(oc, cnt_hbm.at[wid])
    pltpu.sync_copy(os_, sum_hbm.at[wid])


_SC_OUT = [jax.ShapeDtypeStruct((NW, NBIN), jnp.int32),
           jax.ShapeDtypeStruct((NW, NBIN), jnp.float32)]
_SC_SCRATCH = [
    pltpu.VMEM((CHUNK,), jnp.float32),
    pltpu.VMEM((CHUNK,), jnp.float32),
    pltpu.VMEM((2, 16), jnp.float32),
    pltpu.VMEM((1, 16), jnp.int32),
    pltpu.VMEM((16 * NBIN,), jnp.int32),
    pltpu.VMEM((16 * NBIN,), jnp.float32),
    pltpu.VMEM((NBIN,), jnp.int32),
    pltpu.VMEM((NBIN,), jnp.float32),
    pltpu.SemaphoreType.DMA,
    pltpu.SemaphoreType.DMA,
]


@functools.partial(
    pl.kernel,
    mesh=_MESH,
    compiler_params=pltpu.CompilerParams(needs_layout_passes=False),
    out_type=_SC_OUT,
    scratch_types=_SC_SCRATCH,
)
def _sc_hist_coarse(loss_hbm, cnt_hbm, sum_hbm, *rest):
    _hist_common(False, loss_hbm, None, cnt_hbm, sum_hbm, *rest)


@functools.partial(
    pl.kernel,
    mesh=_MESH,
    compiler_params=pltpu.CompilerParams(needs_layout_passes=False),
    out_type=_SC_OUT,
    scratch_types=_SC_SCRATCH,
)
def _sc_hist_fine(loss_hbm, lo_hbm, invw_hbm, bsel_hbm, cnt_hbm, sum_hbm,
                  *rest):
    _hist_common(True, loss_hbm, (lo_hbm, invw_hbm, bsel_hbm),
                 cnt_hbm, sum_hbm, *rest)


def _suffix_sum(x):
    row = lax.broadcasted_iota(jnp.int32, (NBIN, NBIN), 0)
    col = lax.broadcasted_iota(jnp.int32, (NBIN, NBIN), 1)
    tri = (row >= col).astype(jnp.float32)
    return jnp.dot(x.reshape(1, NBIN), tri,
                   preferred_element_type=jnp.float32).reshape(NBIN)


def _select_body(cnt_ref, sum_ref, out_ref):
    c = jnp.sum(cnt_ref[...], axis=0)
    s = jnp.sum(sum_ref[...], axis=0)
    cg = _suffix_sum(c.astype(jnp.float32))
    bsel = jnp.sum((cg >= K).astype(jnp.int32)) - 1
    bins = lax.iota(jnp.int32, NBIN)
    above = bins > bsel
    c_above = jnp.sum(jnp.where(above, c, 0))
    s_above = jnp.sum(jnp.where(above, s, 0.0))
    total = jnp.sum(s)
    vals = [bsel.astype(jnp.float32) * W_COARSE,
            jnp.float32(NBIN / W_COARSE),
            bsel.astype(jnp.float32),
            c_above.astype(jnp.float32),
            s_above,
            total]
    row = lax.broadcasted_iota(jnp.int32, (8, 128), 0)
    col = lax.broadcasted_iota(jnp.int32, (8, 128), 1)
    o = jnp.zeros((8, 128), jnp.float32)
    for j, v in enumerate(vals):
        o = jnp.where((row == 0) & (col == j), v, o)
    out_ref[...] = o


def _select(cnt, sums):
    return pl.pallas_call(
        _select_body,
        out_shape=jax.ShapeDtypeStruct((8, 128), jnp.float32),
    )(cnt, sums)


def _final_body(prm_ref, cnt_ref, sum_ref, out_ref):
    lo = prm_ref[0, 0]
    c_above = prm_ref[0, 3]
    s_above = prm_ref[0, 4]
    total = prm_ref[0, 5]
    fc = jnp.sum(cnt_ref[...], axis=0)
    fs = jnp.sum(sum_ref[...], axis=0)
    cgf = _suffix_sum(fc.astype(jnp.float32))
    fsel = jnp.sum((c_above + cgf >= K).astype(jnp.int32)) - 1
    bins = lax.iota(jnp.int32, NBIN)
    above = bins > fsel
    n_above_f = jnp.sum(jnp.where(above, fc, 0)).astype(jnp.float32)
    s_above_f = jnp.sum(jnp.where(above, fs, 0.0))
    needed = K - c_above - n_above_f
    w_f = W_COARSE / NBIN
    t_est = lo + (fsel.astype(jnp.float32) + 0.5) * w_f
    topk_sum = s_above + s_above_f + needed * t_est
    loss_total = total / (N + 1e-12) + topk_sum / K
    out_ref[...] = jnp.full((1, 1), loss_total)


def _final(prm, cnt, sums):
    return pl.pallas_call(
        _final_body,
        out_shape=jax.ShapeDtypeStruct((1, 1), jnp.float32),
    )(prm, cnt, sums)


def kernel(pred, gt):
    pred2d = pred.reshape(8192, 512)
    gt2d = gt.reshape(8192, 512)
    loss = _bce_loss(pred2d, gt2d)
    loss_flat = loss.reshape(N)

    cc, cs = _sc_hist_coarse(loss_flat)

    prm = _select(cc, cs)

    lo1 = jnp.full((16,), prm[0, 0])
    invw1 = jnp.full((16,), prm[0, 1])
    bsel1 = jnp.full((16,), prm[0, 2].astype(jnp.int32))
    fc, fs = _sc_hist_fine(loss_flat, lo1, invw1, bsel1)

    out = _final(prm, fc, fs)
    return out[0, 0]


# speedup vs baseline: 37.5950x; 1.0907x over previous
import functools

import jax
import jax.numpy as jnp
from jax import lax
from jax.experimental import pallas as pl
from jax.experimental.pallas import tpu as pltpu, tpu_sc as plsc

N = 16 * 1 * 512 * 512
K = int(N * 0.3)
NBIN = 1024
LOSS_MAX = 100.0
C_SCALE = float(NBIN) / LOSS_MAX
W_COARSE = LOSS_MAX / NBIN

NW = 32
PER_W = N // NW
CHUNK = 8192
NCHUNK = PER_W // CHUNK
GROUPS = CHUNK // 16

_MESH = plsc.VectorSubcoreMesh(core_axis_name="c", subcore_axis_name="s")


def _loss_body(pred_ref, gt_ref, loss_ref):
    p = pred_ref[...]
    g = gt_ref[...]
    log_p = jnp.maximum(jnp.log(p), -100.0)
    log_1mp = jnp.maximum(jnp.log(1.0 - p), -100.0)
    loss_ref[...] = -(g * log_p + (1.0 - g) * log_1mp)


def _bce_loss(pred2d, gt2d):
    rows = pred2d.shape[0]
    blk = rows // 8
    return pl.pallas_call(
        _loss_body,
        grid=(8,),
        in_specs=[pl.BlockSpec((blk, 512), lambda i: (i, 0)),
                  pl.BlockSpec((blk, 512), lambda i: (i, 0))],
        out_specs=pl.BlockSpec((blk, 512), lambda i: (i, 0)),
        out_shape=jax.ShapeDtypeStruct((rows, 512), jnp.float32),
    )(pred2d, gt2d)


def _hist_common(masked, loss_hbm, prm, cnt_hbm, sum_hbm,
                 buf0, buf1, prm_f, prm_i, hc, hs, oc, os_, sem0, sem1):
    wid = lax.axis_index("s") * 2 + lax.axis_index("c")
    row0 = wid * (PER_W // 512)

    if masked:
        lo_hbm, invw_hbm, bsel_hbm = prm
        pltpu.sync_copy(lo_hbm, prm_f.at[0])
        pltpu.sync_copy(invw_hbm, prm_f.at[1])
        pltpu.sync_copy(bsel_hbm, prm_i.at[0])
        lo = prm_f[0]
        invw = prm_f[1]
        bsel = prm_i[0]

    zi = jnp.zeros((16,), jnp.int32)
    zf = jnp.zeros((16,), jnp.float32)

    @plsc.parallel_loop(0, NBIN, unroll=8)
    def _zero(g):
        hc[pl.ds(g * 16, 16)] = zi
        hs[pl.ds(g * 16, 16)] = zf

    lane_off = lax.iota(jnp.int32, 16) * NBIN
    ones_i = jnp.ones((16,), jnp.int32)

    bufs = (buf0, buf1)
    sems = (sem0, sem1)
    crows = CHUNK // 512
    pend = [None, None]
    pend[0] = pltpu.async_copy(loss_hbm.at[pl.ds(row0, crows), :], buf0, sem0)
    for c in range(NCHUNK):
        pend[c % 2].wait()
        if c + 1 < NCHUNK:
            pend[(c + 1) % 2] = pltpu.async_copy(
                loss_hbm.at[pl.ds(row0 + (c + 1) * crows, crows), :],
                bufs[(c + 1) % 2], sems[(c + 1) % 2])
        buf = bufs[c % 2]

        @plsc.parallel_loop(0, GROUPS, unroll=8)
        def _group(g):
            v = buf[lax.shift_right_logical(g, 5),
                    pl.ds(lax.bitwise_and(g, 31) * 16, 16)]
            if masked:
                cidx = jnp.clip((v * C_SCALE).astype(jnp.int32), 0, NBIN - 1)
                mask = cidx == bsel
                fidx = jnp.clip(((v - lo) * invw).astype(jnp.int32),
                                0, NBIN - 1)
                addr = fidx + lane_off
                plsc.addupdate_scatter(hc, [addr], ones_i, mask=mask)
                plsc.addupdate_scatter(hs, [addr], v, mask=mask)
            else:
                addr = jnp.clip((v * C_SCALE).astype(jnp.int32),
                                0, NBIN - 1) + lane_off
                plsc.addupdate_scatter(hc, [addr], ones_i)
                plsc.addupdate_scatter(hs, [addr], v)

    @plsc.parallel_loop(0, NBIN // 16, unroll=2)
    def _red(g):
        ac = hc[pl.ds(g * 16, 16)]
        af = hs[pl.ds(g * 16, 16)]
        for l in range(1, 16):
            ac = ac + hc[pl.ds(l * NBIN + g * 16, 16)]
            af = af + hs[pl.ds(l * NBIN + g * 16, 16)]
        oc[pl.ds(g * 16, 16)] = ac
        os_[pl.ds(g * 16, 16)] = af

    pltpu.sync_copy(oc, cnt_hbm.at[pl.ds(wid * NBIN, NBIN)])
    pltpu.sync_copy(os_, sum_hbm.at[pl.ds(wid * NBIN, NBIN)])


_SC_OUT = [jax.ShapeDtypeStruct((NW * NBIN,), jnp.int32),
           jax.ShapeDtypeStruct((NW * NBIN,), jnp.float32)]
_SC_SCRATCH = [
    pltpu.VMEM((CHUNK // 512, 512), jnp.float32),
    pltpu.VMEM((CHUNK // 512, 512), jnp.float32),
    pltpu.VMEM((2, 16), jnp.float32),
    pltpu.VMEM((1, 16), jnp.int32),
    pltpu.VMEM((16 * NBIN,), jnp.int32),
    pltpu.VMEM((16 * NBIN,), jnp.float32),
    pltpu.VMEM((NBIN,), jnp.int32),
    pltpu.VMEM((NBIN,), jnp.float32),
    pltpu.SemaphoreType.DMA,
    pltpu.SemaphoreType.DMA,
]


_SC_PARAMS = pltpu.CompilerParams(needs_layout_passes=False,
                                  use_tc_tiling_on_sc=True)


@functools.partial(
    pl.kernel,
    mesh=_MESH,
    compiler_params=_SC_PARAMS,
    out_type=_SC_OUT,
    scratch_types=_SC_SCRATCH,
)
def _sc_hist_coarse(loss_hbm, cnt_hbm, sum_hbm, *rest):
    _hist_common(False, loss_hbm, None, cnt_hbm, sum_hbm, *rest)


@functools.partial(
    pl.kernel,
    mesh=_MESH,
    compiler_params=_SC_PARAMS,
    out_type=_SC_OUT,
    scratch_types=_SC_SCRATCH,
)
def _sc_hist_fine(loss_hbm, lo_hbm, invw_hbm, bsel_hbm, cnt_hbm, sum_hbm,
                  *rest):
    _hist_common(True, loss_hbm, (lo_hbm, invw_hbm, bsel_hbm),
                 cnt_hbm, sum_hbm, *rest)


def _suffix_sum(x):
    row = lax.broadcasted_iota(jnp.int32, (NBIN, NBIN), 0)
    col = lax.broadcasted_iota(jnp.int32, (NBIN, NBIN), 1)
    tri = (row >= col).astype(jnp.float32)
    return jnp.dot(x.reshape(1, NBIN), tri,
                   preferred_element_type=jnp.float32).reshape(NBIN)


def _select_body(cnt_ref, sum_ref, out_ref):
    c = jnp.sum(cnt_ref[...], axis=0)
    s = jnp.sum(sum_ref[...], axis=0)
    cg = _suffix_sum(c.astype(jnp.float32))
    bsel = jnp.sum((cg >= K).astype(jnp.int32)) - 1
    bins = lax.iota(jnp.int32, NBIN)
    above = bins > bsel
    c_above = jnp.sum(jnp.where(above, c, 0))
    s_above = jnp.sum(jnp.where(above, s, 0.0))
    total = jnp.sum(s)
    vals = [bsel.astype(jnp.float32) * W_COARSE,
            jnp.float32(NBIN / W_COARSE),
            bsel.astype(jnp.float32),
            c_above.astype(jnp.float32),
            s_above,
            total]
    row = lax.broadcasted_iota(jnp.int32, (8, 128), 0)
    col = lax.broadcasted_iota(jnp.int32, (8, 128), 1)
    o = jnp.zeros((8, 128), jnp.float32)
    for j, v in enumerate(vals):
        o = jnp.where((row == 0) & (col == j), v, o)
    out_ref[...] = o


def _select(cnt, sums):
    return pl.pallas_call(
        _select_body,
        out_shape=jax.ShapeDtypeStruct((8, 128), jnp.float32),
    )(cnt, sums)


def _final_body(prm_ref, cnt_ref, sum_ref, out_ref):
    lo = prm_ref[0, 0]
    c_above = prm_ref[0, 3]
    s_above = prm_ref[0, 4]
    total = prm_ref[0, 5]
    fc = jnp.sum(cnt_ref[...], axis=0)
    fs = jnp.sum(sum_ref[...], axis=0)
    cgf = _suffix_sum(fc.astype(jnp.float32))
    fsel = jnp.sum((c_above + cgf >= K).astype(jnp.int32)) - 1
    bins = lax.iota(jnp.int32, NBIN)
    above = bins > fsel
    n_above_f = jnp.sum(jnp.where(above, fc, 0)).astype(jnp.float32)
    s_above_f = jnp.sum(jnp.where(above, fs, 0.0))
    needed = K - c_above - n_above_f
    w_f = W_COARSE / NBIN
    t_est = lo + (fsel.astype(jnp.float32) + 0.5) * w_f
    topk_sum = s_above + s_above_f + needed * t_est
    loss_total = total / (N + 1e-12) + topk_sum / K
    out_ref[...] = jnp.full((1, 1), loss_total)


def _final(prm, cnt, sums):
    return pl.pallas_call(
        _final_body,
        out_shape=jax.ShapeDtypeStruct((1, 1), jnp.float32),
    )(prm, cnt, sums)


def kernel(pred, gt):
    pred2d = pred.reshape(8192, 512)
    gt2d = gt.reshape(8192, 512)
    loss = _bce_loss(pred2d, gt2d)

    cc, cs = _sc_hist_coarse(loss)

    prm = _select(cc.reshape(NW, NBIN), cs.reshape(NW, NBIN))

    lo1 = jnp.full((16,), prm[0, 0])
    invw1 = jnp.full((16,), prm[0, 1])
    bsel1 = jnp.full((16,), prm[0, 2].astype(jnp.int32))
    fc, fs = _sc_hist_fine(loss, lo1, invw1, bsel1)

    out = _final(prm, fc.reshape(NW, NBIN), fs.reshape(NW, NBIN))
    return out[0, 0]


# speedup vs baseline: 41.9248x; 1.1152x over previous
import functools

import jax
import jax.numpy as jnp
from jax import lax
from jax.experimental import pallas as pl
from jax.experimental.pallas import tpu as pltpu, tpu_sc as plsc

N = 16 * 1 * 512 * 512
K = int(N * 0.3)
NBIN = 1024
LOSS_MAX = 100.0
C_SCALE = float(NBIN) / LOSS_MAX
W_COARSE = LOSS_MAX / NBIN

LSTRIDE = NBIN + 17
NW = 32
PER_W = N // NW
CHUNK = 8192
NCHUNK = PER_W // CHUNK
GROUPS = CHUNK // 16

_MESH = plsc.VectorSubcoreMesh(core_axis_name="c", subcore_axis_name="s")


def _loss_body(pred_ref, gt_ref, loss_ref):
    p = pred_ref[...]
    g = gt_ref[...]
    log_p = jnp.maximum(jnp.log(p), -100.0)
    log_1mp = jnp.maximum(jnp.log(1.0 - p), -100.0)
    loss_ref[...] = -(g * log_p + (1.0 - g) * log_1mp)


def _bce_loss(pred2d, gt2d):
    rows = pred2d.shape[0]
    blk = rows // 8
    return pl.pallas_call(
        _loss_body,
        grid=(8,),
        in_specs=[pl.BlockSpec((blk, 512), lambda i: (i, 0)),
                  pl.BlockSpec((blk, 512), lambda i: (i, 0))],
        out_specs=pl.BlockSpec((blk, 512), lambda i: (i, 0)),
        out_shape=jax.ShapeDtypeStruct((rows, 512), jnp.float32),
    )(pred2d, gt2d)


def _hist_common(masked, loss_hbm, prm, cnt_hbm, sum_hbm,
                 buf0, buf1, prm_f, prm_i, hc, hs, oc, os_, sem0, sem1):
    wid = lax.axis_index("s") * 2 + lax.axis_index("c")
    row0 = wid * (PER_W // 512)

    if masked:
        lo_hbm, invw_hbm, bsel_hbm = prm
        pltpu.sync_copy(lo_hbm, prm_f.at[0])
        pltpu.sync_copy(invw_hbm, prm_f.at[1])
        pltpu.sync_copy(bsel_hbm, prm_i.at[0])
        lo = prm_f[0]
        invw = prm_f[1]
        bsel = prm_i[0]

    zi = jnp.zeros((16,), jnp.int32)
    zf = jnp.zeros((16,), jnp.float32)

    @plsc.parallel_loop(0, (16 * LSTRIDE) // 16, unroll=8)
    def _zero(g):
        hc[pl.ds(g * 16, 16)] = zi
        hs[pl.ds(g * 16, 16)] = zf

    lane_off = lax.iota(jnp.int32, 16) * LSTRIDE
    ones_i = jnp.ones((16,), jnp.int32)

    bufs = (buf0, buf1)
    sems = (sem0, sem1)
    crows = CHUNK // 512
    pend = [None, None]
    pend[0] = pltpu.async_copy(loss_hbm.at[pl.ds(row0, crows), :], buf0, sem0)
    for c in range(NCHUNK):
        pend[c % 2].wait()
        if c + 1 < NCHUNK:
            pend[(c + 1) % 2] = pltpu.async_copy(
                loss_hbm.at[pl.ds(row0 + (c + 1) * crows, crows), :],
                bufs[(c + 1) % 2], sems[(c + 1) % 2])
        buf = bufs[c % 2]

        @plsc.parallel_loop(0, GROUPS, unroll=8)
        def _group(g):
            v = buf[lax.shift_right_logical(g, 5),
                    pl.ds(lax.bitwise_and(g, 31) * 16, 16)]
            if masked:
                cidx = jnp.clip((v * C_SCALE).astype(jnp.int32), 0, NBIN - 1)
                mask = cidx == bsel
                fidx = jnp.clip(((v - lo) * invw).astype(jnp.int32),
                                0, NBIN - 1)
                addr = fidx + lane_off
                plsc.addupdate_scatter(hc, [addr], ones_i, mask=mask)
                plsc.addupdate_scatter(hs, [addr], v, mask=mask)
            else:
                addr = jnp.clip((v * C_SCALE).astype(jnp.int32),
                                0, NBIN - 1) + lane_off
                plsc.addupdate_scatter(hc, [addr], ones_i)
                plsc.addupdate_scatter(hs, [addr], v)

    @plsc.parallel_loop(0, NBIN // 16, unroll=2)
    def _red(g):
        ac = hc[pl.ds(g * 16, 16)]
        af = hs[pl.ds(g * 16, 16)]
        for l in range(1, 16):
            ac = ac + hc[pl.ds(l * LSTRIDE + g * 16, 16)]
            af = af + hs[pl.ds(l * LSTRIDE + g * 16, 16)]
        oc[pl.ds(g * 16, 16)] = ac
        os_[pl.ds(g * 16, 16)] = af

    pltpu.sync_copy(oc, cnt_hbm.at[pl.ds(wid * NBIN, NBIN)])
    pltpu.sync_copy(os_, sum_hbm.at[pl.ds(wid * NBIN, NBIN)])


_SC_OUT = [jax.ShapeDtypeStruct((NW * NBIN,), jnp.int32),
           jax.ShapeDtypeStruct((NW * NBIN,), jnp.float32)]
_SC_SCRATCH = [
    pltpu.VMEM((CHUNK // 512, 512), jnp.float32),
    pltpu.VMEM((CHUNK // 512, 512), jnp.float32),
    pltpu.VMEM((2, 16), jnp.float32),
    pltpu.VMEM((1, 16), jnp.int32),
    pltpu.VMEM((16 * LSTRIDE,), jnp.int32),
    pltpu.VMEM((16 * LSTRIDE,), jnp.float32),
    pltpu.VMEM((NBIN,), jnp.int32),
    pltpu.VMEM((NBIN,), jnp.float32),
    pltpu.SemaphoreType.DMA,
    pltpu.SemaphoreType.DMA,
]


_SC_PARAMS = pltpu.CompilerParams(needs_layout_passes=False,
                                  use_tc_tiling_on_sc=True)


@functools.partial(
    pl.kernel,
    mesh=_MESH,
    compiler_params=_SC_PARAMS,
    out_type=_SC_OUT,
    scratch_types=_SC_SCRATCH,
)
def _sc_hist_coarse(loss_hbm, cnt_hbm, sum_hbm, *rest):
    _hist_common(False, loss_hbm, None, cnt_hbm, sum_hbm, *rest)


@functools.partial(
    pl.kernel,
    mesh=_MESH,
    compiler_params=_SC_PARAMS,
    out_type=_SC_OUT,
    scratch_types=_SC_SCRATCH,
)
def _sc_hist_fine(loss_hbm, lo_hbm, invw_hbm, bsel_hbm, cnt_hbm, sum_hbm,
                  *rest):
    _hist_common(True, loss_hbm, (lo_hbm, invw_hbm, bsel_hbm),
                 cnt_hbm, sum_hbm, *rest)


def _suffix_sum(x):
    row = lax.broadcasted_iota(jnp.int32, (NBIN, NBIN), 0)
    col = lax.broadcasted_iota(jnp.int32, (NBIN, NBIN), 1)
    tri = (row >= col).astype(jnp.float32)
    return jnp.dot(x.reshape(1, NBIN), tri,
                   preferred_element_type=jnp.float32).reshape(NBIN)


def _select_body(cnt_ref, sum_ref, out_ref):
    c = jnp.sum(cnt_ref[...], axis=0)
    s = jnp.sum(sum_ref[...], axis=0)
    cg = _suffix_sum(c.astype(jnp.float32))
    bsel = jnp.sum((cg >= K).astype(jnp.int32)) - 1
    bins = lax.iota(jnp.int32, NBIN)
    above = bins > bsel
    c_above = jnp.sum(jnp.where(above, c, 0))
    s_above = jnp.sum(jnp.where(above, s, 0.0))
    total = jnp.sum(s)
    vals = [bsel.astype(jnp.float32) * W_COARSE,
            jnp.float32(NBIN / W_COARSE),
            bsel.astype(jnp.float32),
            c_above.astype(jnp.float32),
            s_above,
            total]
    row = lax.broadcasted_iota(jnp.int32, (8, 128), 0)
    col = lax.broadcasted_iota(jnp.int32, (8, 128), 1)
    o = jnp.zeros((8, 128), jnp.float32)
    for j, v in enumerate(vals):
        o = jnp.where((row == 0) & (col == j), v, o)
    out_ref[...] = o


def _select(cnt, sums):
    return pl.pallas_call(
        _select_body,
        out_shape=jax.ShapeDtypeStruct((8, 128), jnp.float32),
    )(cnt, sums)


def _final_body(prm_ref, cnt_ref, sum_ref, out_ref):
    lo = prm_ref[0, 0]
    c_above = prm_ref[0, 3]
    s_above = prm_ref[0, 4]
    total = prm_ref[0, 5]
    fc = jnp.sum(cnt_ref[...], axis=0)
    fs = jnp.sum(sum_ref[...], axis=0)
    cgf = _suffix_sum(fc.astype(jnp.float32))
    fsel = jnp.sum((c_above + cgf >= K).astype(jnp.int32)) - 1
    bins = lax.iota(jnp.int32, NBIN)
    above = bins > fsel
    n_above_f = jnp.sum(jnp.where(above, fc, 0)).astype(jnp.float32)
    s_above_f = jnp.sum(jnp.where(above, fs, 0.0))
    needed = K - c_above - n_above_f
    w_f = W_COARSE / NBIN
    t_est = lo + (fsel.astype(jnp.float32) + 0.5) * w_f
    topk_sum = s_above + s_above_f + needed * t_est
    loss_total = total / (N + 1e-12) + topk_sum / K
    out_ref[...] = jnp.full((1, 1), loss_total)


def _final(prm, cnt, sums):
    return pl.pallas_call(
        _final_body,
        out_shape=jax.ShapeDtypeStruct((1, 1), jnp.float32),
    )(prm, cnt, sums)


def kernel(pred, gt):
    pred2d = pred.reshape(8192, 512)
    gt2d = gt.reshape(8192, 512)
    loss = _bce_loss(pred2d, gt2d)

    cc, cs = _sc_hist_coarse(loss)

    prm = _select(cc.reshape(NW, NBIN), cs.reshape(NW, NBIN))

    lo1 = jnp.full((16,), prm[0, 0])
    invw1 = jnp.full((16,), prm[0, 1])
    bsel1 = jnp.full((16,), prm[0, 2].astype(jnp.int32))
    fc, fs = _sc_hist_fine(loss, lo1, invw1, bsel1)

    out = _final(prm, fc.reshape(NW, NBIN), fs.reshape(NW, NBIN))
    return out[0, 0]
